# baseline (speedup 1.0000x reference)
import functools
import math

import jax
import jax.numpy as jnp
from jax import lax
from jax.experimental import pallas as pl
from jax.experimental.pallas import tpu as pltpu
from jax.experimental.pallas import tpu_sc as plsc

_H = 8
_DH = 16
_D = 128


def _mm_bias_kernel(x_ref, w_ref, b_ref, o_ref):
    o_ref[...] = (
        lax.dot_general(x_ref[...], w_ref[...], (((1,), (1,)), ((), ())),
                        preferred_element_type=jnp.float32)
        + b_ref[...]
    )


def _matmul_bias(x, w, b, block):
    m, kdim = x.shape
    o = w.shape[0]
    return pl.pallas_call(
        _mm_bias_kernel,
        grid=(m // block,),
        in_specs=[
            pl.BlockSpec((block, kdim), lambda i: (i, 0)),
            pl.BlockSpec((o, kdim), lambda i: (0, 0)),
            pl.BlockSpec((1, o), lambda i: (0, 0)),
        ],
        out_specs=pl.BlockSpec((block, o), lambda i: (i, 0)),
        out_shape=jax.ShapeDtypeStruct((m, o), jnp.float32),
    )(x, w, b.reshape(1, o))


def _mm_cat2_kernel(x1_ref, x2_ref, w_ref, b_ref, o_ref):
    w = w_ref[...]
    a = lax.dot_general(x1_ref[...], w[:, :_D], (((1,), (1,)), ((), ())),
                        preferred_element_type=jnp.float32)
    c = lax.dot_general(x2_ref[...], w[:, _D:], (((1,), (1,)), ((), ())),
                        preferred_element_type=jnp.float32)
    o_ref[...] = a + c + b_ref[...]


def _matmul_cat2(x1, x2, w, b, block):
    m, kdim = x1.shape
    o = w.shape[0]
    return pl.pallas_call(
        _mm_cat2_kernel,
        grid=(m // block,),
        in_specs=[
            pl.BlockSpec((block, kdim), lambda i: (i, 0)),
            pl.BlockSpec((block, kdim), lambda i: (i, 0)),
            pl.BlockSpec((o, 2 * kdim), lambda i: (0, 0)),
            pl.BlockSpec((1, o), lambda i: (0, 0)),
        ],
        out_specs=pl.BlockSpec((block, o), lambda i: (i, 0)),
        out_shape=jax.ShapeDtypeStruct((m, o), jnp.float32),
    )(x1, x2, w, b.reshape(1, o))


def _scatter_softmax(x, seg, num_seg):
    m = jax.ops.segment_max(x, seg, num_segments=num_seg)
    m = jnp.where(jnp.isfinite(m), m, 0.0)
    e = jnp.exp(x - m[seg])
    ssum = jax.ops.segment_sum(e, seg, num_segments=num_seg)
    return e / ssum[seg]


def kernel(f_node, deg_matrix, dist_matrix, f_bond, bond_idx, query_idx,
           key_idx, attention_bond_idx, bond_split, Wq_w, Wq_b, Wk_w, Wk_b,
           Wv_w, Wv_b, out_w, out_b, deg_emb, dist_emb, bond_emb_w,
           bond_emb_b, bond_update_w, bond_update_b):
    n = f_node.shape[0]
    nb = f_bond.shape[0]

    q = _matmul_bias(f_node, Wq_w, Wq_b, 400).reshape(n, 2 * _H, _DH)
    k = _matmul_bias(f_node, Wk_w, Wk_b, 400).reshape(n, 2 * _H, _DH)
    v = _matmul_bias(f_node, Wv_w, Wv_b, 400).reshape(n, 2 * _H, _DH)
    bond_proj = _matmul_bias(f_bond, bond_emb_w, bond_emb_b, 640)
    attention_bond = bond_proj[:, :_D].reshape(nb, _H, _DH)
    value_bond = bond_proj[:, _D:].reshape(nb, _H, _DH)

    f_deg = deg_emb[deg_matrix].reshape(n, _H, _DH)
    f_dist = dist_emb[dist_matrix].reshape(-1, _H, _DH)

    q_local, q_global = q[:, :_H], q[:, _H:]
    k_local, k_global = k[:, :_H], k[:, _H:]
    v_local, v_global = v[:, :_H], v[:, _H:]

    score_local = q_local[bond_idx[1]] * k_local[bond_idx[0]]
    score_local = score_local.at[:nb].add(attention_bond)
    score_local = score_local.sum(axis=-1) / math.sqrt(_DH)
    score_local = _scatter_softmax(score_local, bond_idx[1], n)
    local_out = v_local[bond_idx[0]]
    local_out = local_out.at[:nb].add(value_bond)
    local_out = local_out * score_local[..., None]
    local_out = jax.ops.segment_sum(local_out, bond_idx[1], num_segments=n)
    local_out = local_out.reshape(-1, _H * _DH)

    score_global = q_global[query_idx] * (k_global + f_deg)[key_idx]
    score_global = score_global.at[attention_bond_idx].add(attention_bond)
    score_global = score_global + f_dist
    score_global = score_global.sum(axis=-1) / math.sqrt(_DH)
    score_global = _scatter_softmax(score_global, query_idx, n)
    global_out = v_global[key_idx] * score_global[..., None]
    global_out = jax.ops.segment_sum(global_out, query_idx, num_segments=n)
    global_out = global_out.reshape(-1, _H * _DH)

    out = _matmul_cat2(local_out, global_out, out_w, out_b, 400)
    gath = out[bond_idx[0, :nb]]
    bond_out = _matmul_cat2(f_bond, gath, bond_update_w, bond_update_b, 640)
    return out, bond_out


# speedup vs baseline: 11.4328x; 11.4328x over previous
import functools
import math

import jax
import jax.numpy as jnp
from jax import lax
from jax.experimental import pallas as pl
from jax.experimental.pallas import tpu as pltpu
from jax.experimental.pallas import tpu_sc as plsc

_H = 8
_DH = 16
_D = 128
_NC = 2
_NS = 16
_NW = _NC * _NS
_L = 16
_G = 80


def _mm_bias_kernel(scale, x_ref, w_ref, b_ref, o_ref):
    o = (
        lax.dot_general(x_ref[...], w_ref[...], (((1,), (1,)), ((), ())),
                        preferred_element_type=jnp.float32)
        + b_ref[...]
    )
    o_ref[...] = o * scale


def _matmul_bias(x, w, b, block, scale=1.0):
    m, kdim = x.shape
    o = w.shape[0]
    return pl.pallas_call(
        functools.partial(_mm_bias_kernel, scale),
        grid=(m // block,),
        in_specs=[
            pl.BlockSpec((block, kdim), lambda i: (i, 0)),
            pl.BlockSpec((o, kdim), lambda i: (0, 0)),
            pl.BlockSpec((1, o), lambda i: (0, 0)),
        ],
        out_specs=pl.BlockSpec((block, o), lambda i: (i, 0)),
        out_shape=jax.ShapeDtypeStruct((m, o), jnp.float32),
    )(x, w, b.reshape(1, o))


def _mm_cat2_kernel(x1_ref, x2_ref, w_ref, b_ref, o_ref):
    w = w_ref[...]
    a = lax.dot_general(x1_ref[...], w[:, :_D], (((1,), (1,)), ((), ())),
                        preferred_element_type=jnp.float32)
    c = lax.dot_general(x2_ref[...], w[:, _D:], (((1,), (1,)), ((), ())),
                        preferred_element_type=jnp.float32)
    o_ref[...] = a + c + b_ref[...]


def _matmul_cat2(x1, x2, w, b, block):
    m, kdim = x1.shape
    o = w.shape[0]
    return pl.pallas_call(
        _mm_cat2_kernel,
        grid=(m // block,),
        in_specs=[
            pl.BlockSpec((block, kdim), lambda i: (i, 0)),
            pl.BlockSpec((block, kdim), lambda i: (i, 0)),
            pl.BlockSpec((o, 2 * kdim), lambda i: (0, 0)),
            pl.BlockSpec((1, o), lambda i: (0, 0)),
        ],
        out_specs=pl.BlockSpec((block, o), lambda i: (i, 0)),
        out_shape=jax.ShapeDtypeStruct((m, o), jnp.float32),
    )(x1, x2, w, b.reshape(1, o))


def _mm_sum2cat_kernel(l_ref, g_ref, w_ref, b_ref, o_ref):
    lsum = l_ref[0] + l_ref[1]
    gsum = g_ref[0] + g_ref[1]
    w = w_ref[...]
    a = lax.dot_general(lsum, w[:, :_D], (((1,), (1,)), ((), ())),
                        preferred_element_type=jnp.float32)
    c = lax.dot_general(gsum, w[:, _D:], (((1,), (1,)), ((), ())),
                        preferred_element_type=jnp.float32)
    o_ref[...] = a + c + b_ref[...]


def _matmul_sum2cat(lp, gp, w, b, block):
    _, m, kdim = lp.shape
    o = w.shape[0]
    return pl.pallas_call(
        _mm_sum2cat_kernel,
        grid=(m // block,),
        in_specs=[
            pl.BlockSpec((2, block, kdim), lambda i: (0, i, 0)),
            pl.BlockSpec((2, block, kdim), lambda i: (0, i, 0)),
            pl.BlockSpec((o, 2 * kdim), lambda i: (0, 0)),
            pl.BlockSpec((1, o), lambda i: (0, 0)),
        ],
        out_specs=pl.BlockSpec((block, o), lambda i: (i, 0)),
        out_shape=jax.ShapeDtypeStruct((m, o), jnp.float32),
    )(lp, gp, w, b.reshape(1, o))


def _keff_kernel(k_ref, deg_ref, demb_ref, kl_ref, kg_ref):
    k = k_ref[...]
    kl_ref[...] = k[:, :_D]
    deg = deg_ref[...]
    onehot = (deg == lax.broadcasted_iota(jnp.int32, deg.shape[:1] + (17,), 1)
              ).astype(jnp.float32)
    fdeg = lax.dot_general(onehot, demb_ref[...], (((1,), (0,)), ((), ())),
                           preferred_element_type=jnp.float32)
    kg_ref[...] = k[:, _D:] + fdeg


def _keff(k, deg, demb, block):
    m = k.shape[0]
    return pl.pallas_call(
        _keff_kernel,
        grid=(m // block,),
        in_specs=[
            pl.BlockSpec((block, 2 * _D), lambda i: (i, 0)),
            pl.BlockSpec((block, 1), lambda i: (i, 0)),
            pl.BlockSpec((17, _D), lambda i: (0, 0)),
        ],
        out_specs=[
            pl.BlockSpec((block, _D), lambda i: (i, 0)),
            pl.BlockSpec((block, _D), lambda i: (i, 0)),
        ],
        out_shape=[
            jax.ShapeDtypeStruct((m, _D), jnp.float32),
            jax.ShapeDtypeStruct((m, _D), jnp.float32),
        ],
    )(k, deg.reshape(m, 1), demb)


def _bondproj_kernel(x_ref, w_ref, b_ref, vb_ref, abs_ref):
    proj = (
        lax.dot_general(x_ref[...], w_ref[...], (((1,), (1,)), ((), ())),
                        preferred_element_type=jnp.float32)
        + b_ref[...]
    )
    vb_ref[...] = proj[:, _D:]
    blockdiag = (
        lax.broadcasted_iota(jnp.int32, (_D, _H), 0) // _DH
        == lax.broadcasted_iota(jnp.int32, (_D, _H), 1)
    ).astype(jnp.float32)
    abs_ref[...] = lax.dot_general(
        proj[:, :_D], blockdiag, (((1,), (0,)), ((), ())),
        preferred_element_type=jnp.float32) * 0.25


def _bondproj(f_bond, w, b, block):
    m = f_bond.shape[0]
    return pl.pallas_call(
        _bondproj_kernel,
        grid=(m // block,),
        in_specs=[
            pl.BlockSpec((block, _D), lambda i: (i, 0)),
            pl.BlockSpec((2 * _D, _D), lambda i: (0, 0)),
            pl.BlockSpec((1, 2 * _D), lambda i: (0, 0)),
        ],
        out_specs=[
            pl.BlockSpec((block, _D), lambda i: (i, 0)),
            pl.BlockSpec((block, _H), lambda i: (i, 0)),
        ],
        out_shape=[
            jax.ShapeDtypeStruct((m, _D), jnp.float32),
            jax.ShapeDtypeStruct((m, _H), jnp.float32),
        ],
    )(f_bond, w, b.reshape(1, 2 * _D))


def _dist_score_kernel(demb_ref, o_ref):
    blockdiag = (
        lax.broadcasted_iota(jnp.int32, (_D, _H), 0) // _DH
        == lax.broadcasted_iota(jnp.int32, (_D, _H), 1)
    ).astype(jnp.float32)
    o_ref[...] = lax.dot_general(
        demb_ref[...], blockdiag, (((1,), (0,)), ((), ())),
        preferred_element_type=jnp.float32) * 0.25


def _dist_score(demb):
    return pl.pallas_call(
        _dist_score_kernel,
        out_shape=jax.ShapeDtypeStruct((demb.shape[0], _H), jnp.float32),
    )(demb)


def _dinv_kernel(d_ref, o_ref):
    dsum = d_ref[0] + d_ref[1]
    mask = lax.broadcasted_iota(jnp.int32, dsum.shape, 1) < _H
    o_ref[...] = jnp.where(mask, 1.0 / dsum, 0.0)


def _dinv(den, block):
    _, m, _ = den.shape
    return pl.pallas_call(
        _dinv_kernel,
        grid=(m // block,),
        in_specs=[pl.BlockSpec((2, block, _D), lambda i: (0, i, 0))],
        out_specs=pl.BlockSpec((block, _D), lambda i: (i, 0)),
        out_shape=jax.ShapeDtypeStruct((m, _D), jnp.float32),
    )(den)


def _sc_mesh():
    return plsc.VectorSubcoreMesh(core_axis_name="c", subcore_axis_name="s")


def _make_pass_a(e_total, nseg):
    epw = e_total // _NW
    nchunk = epw // _G
    rpt = nseg // _NS

    @functools.partial(
        pl.kernel,
        mesh=_sc_mesh(),
        compiler_params=pltpu.CompilerParams(needs_layout_passes=False),
        out_type=[
            jax.ShapeDtypeStruct((e_total * _H,), jnp.float32),
            jax.ShapeDtypeStruct((_NC, nseg, _D), jnp.float32),
        ],
        scratch_types=[
            pltpu.VMEM((_G,), jnp.int32),
            pltpu.VMEM((_G,), jnp.int32),
            pltpu.VMEM((_G,), jnp.int32),
            pltpu.VMEM((_G, _D), jnp.float32),
            pltpu.VMEM((_G, _D), jnp.float32),
            pltpu.VMEM((_G * _H,), jnp.float32),
            pltpu.VMEM((_G * _H,), jnp.float32),
            pltpu.VMEM((_G, _D), jnp.float32),
            pltpu.VMEM((40 * _H,), jnp.float32),
            pltpu.VMEM_SHARED((nseg, _D), jnp.float32),
            pltpu.SemaphoreType.DMA,
            pltpu.SemaphoreType.DMA,
        ],
    )
    def pass_a(q_hbm, k_hbm, dst_hbm, src_hbm, ab_hbm, did_hbm, dtbl_hbm,
               z128_hbm, p_hbm, den_hbm,
               dst_v, src_v, did_v, q_v, k_v, ab_v, p_v, p128_v, dtbl_v,
               den_sh, sem1, sem2):
        c = lax.axis_index("c")
        s = lax.axis_index("s")
        wid = s * _NC + c
        pltpu.sync_copy(dtbl_hbm, dtbl_v)
        pltpu.sync_copy(z128_hbm.at[pl.ds(s * rpt, rpt)],
                        den_sh.at[pl.ds(s * rpt, rpt)])
        pltpu.sync_copy(z128_hbm.at[pl.ds(0, _G)], p128_v)
        plsc.subcore_barrier()

        lane = lax.iota(jnp.int32, _L)

        def chunk(ci, carry):
            base = wid * epw + ci * _G
            pltpu.sync_copy(dst_hbm.at[pl.ds(base, _G)], dst_v)
            pltpu.sync_copy(src_hbm.at[pl.ds(base, _G)], src_v)
            pltpu.sync_copy(did_hbm.at[pl.ds(base, _G)], did_v)
            pltpu.sync_copy(ab_hbm.at[pl.ds(base * _H, _G * _H)], ab_v)
            cp1 = pltpu.async_copy(q_hbm.at[dst_v], q_v, sem1)
            cp2 = pltpu.async_copy(k_hbm.at[src_v], k_v, sem2)
            cp1.wait()
            cp2.wait()
            for g in range(_G // _L):
                row = g * _L + lane
                did = plsc.load_gather(did_v, [row])
                for h in range(_H):
                    hcol = jnp.full((_L,), h, jnp.int32)
                    acc = plsc.load_gather(ab_v, [row * _H + h])
                    acc = acc + plsc.load_gather(dtbl_v, [did * _H + h])
                    for d in range(_DH):
                        col = jnp.full((_L,), h * _DH + d, jnp.int32)
                        qv = plsc.load_gather(q_v, [row, col])
                        kv = plsc.load_gather(k_v, [row, col])
                        acc = acc + qv * kv
                    p = jnp.exp(acc)
                    plsc.store_scatter(p_v, [row * _H + h], p)
                    plsc.store_scatter(p128_v, [row, hcol], p)
            pltpu.sync_copy(p128_v, den_sh.at[dst_v], add=True)
            pltpu.sync_copy(p_v, p_hbm.at[pl.ds(base * _H, _G * _H)])
            return carry

        lax.fori_loop(0, nchunk, chunk, 0)
        plsc.subcore_barrier()
        pltpu.sync_copy(den_sh.at[pl.ds(s * rpt, rpt)],
                        den_hbm.at[c, pl.ds(s * rpt, rpt)])

    return pass_a


def _make_pass_b(e_total, nseg, has_vb):
    epw = e_total // _NW
    nchunk = epw // _G
    rpt = nseg // _NS

    scratch = [
        pltpu.VMEM((_G,), jnp.int32),
        pltpu.VMEM((_G,), jnp.int32),
        pltpu.VMEM((_G * _H,), jnp.float32),
        pltpu.VMEM((_G, _D), jnp.float32),
        pltpu.VMEM((_G, _D), jnp.float32),
        pltpu.VMEM((_G, _D), jnp.float32),
        pltpu.VMEM_SHARED((nseg, _D), jnp.float32),
        pltpu.SemaphoreType.DMA,
        pltpu.SemaphoreType.DMA,
    ]
    if has_vb:
        scratch.insert(6, pltpu.VMEM((_G, _D), jnp.float32))

    @functools.partial(
        pl.kernel,
        mesh=_sc_mesh(),
        compiler_params=pltpu.CompilerParams(needs_layout_passes=False),
        out_type=jax.ShapeDtypeStruct((_NC, nseg, _D), jnp.float32),
        scratch_types=scratch,
    )
    def pass_b(*refs):
        if has_vb:
            (v_hbm, dst_hbm, src_hbm, p_hbm, dinv_hbm, vb_hbm, z128_hbm,
             out_hbm, dst_v, src_v, p_v, di_v, v_v, ct_v, vb_v, acc_sh,
             sem1, sem2) = refs
        else:
            (v_hbm, dst_hbm, src_hbm, p_hbm, dinv_hbm, z128_hbm,
             out_hbm, dst_v, src_v, p_v, di_v, v_v, ct_v, acc_sh,
             sem1, sem2) = refs
        c = lax.axis_index("c")
        s = lax.axis_index("s")
        wid = s * _NC + c
        pltpu.sync_copy(z128_hbm.at[pl.ds(s * rpt, rpt)],
                        acc_sh.at[pl.ds(s * rpt, rpt)])
        plsc.subcore_barrier()

        lane = lax.iota(jnp.int32, _L)

        def chunk(ci, carry):
            base = wid * epw + ci * _G
            pltpu.sync_copy(dst_hbm.at[pl.ds(base, _G)], dst_v)
            pltpu.sync_copy(src_hbm.at[pl.ds(base, _G)], src_v)
            pltpu.sync_copy(p_hbm.at[pl.ds(base * _H, _G * _H)], p_v)
            if has_vb:
                pltpu.sync_copy(vb_hbm.at[pl.ds(base, _G)], vb_v)
            cp1 = pltpu.async_copy(v_hbm.at[src_v], v_v, sem1)
            cp2 = pltpu.async_copy(dinv_hbm.at[dst_v], di_v, sem2)
            cp1.wait()
            cp2.wait()
            for g in range(_G // _L):
                row = g * _L + lane
                for h in range(_H):
                    hcol = jnp.full((_L,), h, jnp.int32)
                    w = (plsc.load_gather(p_v, [row * _H + h])
                         * plsc.load_gather(di_v, [row, hcol]))
                    for d in range(_DH):
                        col = jnp.full((_L,), h * _DH + d, jnp.int32)
                        vv = plsc.load_gather(v_v, [row, col])
                        if has_vb:
                            vv = vv + plsc.load_gather(vb_v, [row, col])
                        plsc.store_scatter(ct_v, [row, col], vv * w)
            pltpu.sync_copy(ct_v, acc_sh.at[dst_v], add=True)
            return carry

        lax.fori_loop(0, nchunk, chunk, 0)
        plsc.subcore_barrier()
        pltpu.sync_copy(acc_sh.at[pl.ds(s * rpt, rpt)],
                        out_hbm.at[c, pl.ds(s * rpt, rpt)])

    return pass_b


def _make_gather_rows(n_rows):
    rpw = n_rows // _NW
    gc = 40
    assert rpw % gc == 0
    nchunk = rpw // gc

    @functools.partial(
        pl.kernel,
        mesh=_sc_mesh(),
        compiler_params=pltpu.CompilerParams(needs_layout_passes=False),
        out_type=jax.ShapeDtypeStruct((n_rows, _D), jnp.float32),
        scratch_types=[
            pltpu.VMEM((gc,), jnp.int32),
            pltpu.VMEM((gc, _D), jnp.float32),
            pltpu.SemaphoreType.DMA,
        ],
    )
    def gather_rows(tbl_hbm, idx_hbm, out_hbm, idx_v, rows_v, sem):
        c = lax.axis_index("c")
        s = lax.axis_index("s")
        wid = s * _NC + c

        def chunk(ci, carry):
            base = wid * rpw + ci * gc
            pltpu.sync_copy(idx_hbm.at[pl.ds(base, gc)], idx_v)
            pltpu.async_copy(tbl_hbm.at[idx_v], rows_v, sem).wait()
            pltpu.sync_copy(rows_v, out_hbm.at[pl.ds(base, gc)])
            return carry

        lax.fori_loop(0, nchunk, chunk, 0)

    return gather_rows


def kernel(f_node, deg_matrix, dist_matrix, f_bond, bond_idx, query_idx,
           key_idx, attention_bond_idx, bond_split, Wq_w, Wq_b, Wk_w, Wk_b,
           Wv_w, Wv_b, out_w, out_b, deg_emb, dist_emb, bond_emb_w,
           bond_emb_b, bond_update_w, bond_update_b):
    n = f_node.shape[0]
    nb = f_bond.shape[0]
    e_local = bond_idx.shape[1]
    e_glob = query_idx.shape[0]
    nsp = ((n + 8 * _NS - 1) // (8 * _NS)) * (8 * _NS)

    q = _matmul_bias(f_node, Wq_w, Wq_b, 400, scale=0.25)
    k = _matmul_bias(f_node, Wk_w, Wk_b, 400)
    v = _matmul_bias(f_node, Wv_w, Wv_b, 400)
    k_local, k_eff_glob = _keff(k, deg_matrix, deg_emb, 400)
    value_bond, ab_score = _bondproj(f_bond, bond_emb_w, bond_emb_b, 640)
    dscore = _dist_score(dist_emb)
    dscore = jnp.pad(dscore, ((0, 7), (0, 0)))

    q_local = q[:, :_D]
    q_glob = q[:, _D:]
    v_local = v[:, :_D]
    v_glob = v[:, _D:]

    ab_pad = jnp.concatenate(
        [ab_score, jnp.zeros((e_local - nb, _H), jnp.float32)]).reshape(-1)
    vb_pad = jnp.concatenate(
        [value_bond, jnp.zeros((e_local - nb, _D), jnp.float32)])
    zero_ids = jnp.zeros((e_local,), jnp.int32)
    zero_tbl = jnp.zeros((40 * _H,), jnp.float32)
    dscore = dscore.reshape(-1)
    z128 = jnp.zeros((nsp, _D), jnp.float32)

    dst_l = bond_idx[1]
    src_l = bond_idx[0]

    pass_a = _make_pass_a(e_local, nsp)
    p_l, den_l = pass_a(q_local, k_local, dst_l, src_l, ab_pad, zero_ids,
                        zero_tbl, z128)
    p_g, den_g = pass_a(q_glob, k_eff_glob, query_idx, key_idx, ab_pad,
                        dist_matrix, dscore, z128)

    dinv_l = _dinv(den_l, nsp // 8)
    dinv_g = _dinv(den_g, nsp // 8)

    lp = _make_pass_b(e_local, nsp, True)(
        v_local, dst_l, src_l, p_l, dinv_l, vb_pad, z128)
    gp = _make_pass_b(e_glob, nsp, False)(
        v_glob, query_idx, key_idx, p_g, dinv_g, z128)

    out_pad = _matmul_sum2cat(lp, gp, out_w, out_b, nsp // 8)
    gath = _make_gather_rows(nb)(out_pad, src_l[:nb])
    bond_out = _matmul_cat2(f_bond, gath, bond_update_w, bond_update_b, 640)
    return out_pad[:n], bond_out
